# Initial kernel scaffold; baseline (speedup 1.0000x reference)
import jax
import jax.numpy as jnp
from jax.experimental import pallas as pl


def kernel(x):
    raise NotImplementedError("write your pallas kernel here")


# speedup vs baseline: 9.8126x; 9.8126x over previous
import jax
import jax.numpy as jnp
from jax.experimental import pallas as pl
from jax.experimental.pallas import tpu as pltpu

_N = 1024
_D = 64
_T = 12
_B = 16
_K = 5


def _graph_body(x_ref, out_ref):
    nsq = jnp.zeros((_N, 1), jnp.float32)
    for t in range(_T):
        xt = x_ref[t, 0]
        nsq = nsq + jnp.sum(xt * xt, axis=1, keepdims=True)
    rinv = jax.lax.rsqrt(nsq)

    acc = jnp.zeros((_N, _N), jnp.float32)
    for t in range(_T):
        xn = x_ref[t, 0] * rinv
        acc = acc + jax.lax.dot_general(
            xn, xn, (((1,), (1,)), ((), ())),
            preferred_element_type=jnp.float32)

    col_i = jax.lax.broadcasted_iota(jnp.int32, (_N, _N), 1)
    work = acc
    direct = jnp.zeros((_N, _N), jnp.float32)
    for _ in range(_K):
        m = jnp.max(work, axis=1, keepdims=True)
        cand = work == m
        jidx = jnp.min(jnp.where(cand, col_i, _N), axis=1, keepdims=True)
        sel = col_i == jidx
        lv = jnp.where(m >= 0, m, 0.01 * m) * 0.5
        direct = direct + jnp.where(sel, lv, 0.0)
        work = jnp.where(sel, -jnp.inf, work)

    out_ref[0] = direct + direct.T


def kernel(x):
    return pl.pallas_call(
        _graph_body,
        grid=(_B,),
        in_specs=[pl.BlockSpec((_T, 1, _N, _D), lambda b: (0, b, 0, 0))],
        out_specs=pl.BlockSpec((1, _N, _N), lambda b: (b, 0, 0)),
        out_shape=jax.ShapeDtypeStruct((_B, _N, _N), jnp.float32),
    )(x)


# speedup vs baseline: 18.0290x; 1.8373x over previous
import jax
import jax.numpy as jnp
from jax.experimental import pallas as pl
from jax.experimental.pallas import tpu as pltpu

_N = 1024
_D = 64
_T = 12
_B = 16
_K = 5


def _graph_body(x_ref, out_ref):
    xcat = jnp.concatenate([x_ref[t, 0] for t in range(_T)], axis=1)
    nsq = jnp.sum(xcat * xcat, axis=1, keepdims=True)
    xn = xcat * jax.lax.rsqrt(nsq)
    acc = jax.lax.dot_general(
        xn, xn, (((1,), (1,)), ((), ())),
        preferred_element_type=jnp.float32)

    work = acc
    m = jnp.zeros((_N, 1), jnp.float32)
    for _ in range(_K):
        m = jnp.max(work, axis=1, keepdims=True)
        work = jnp.where(work >= m, -jnp.inf, work)
    sel = acc >= m
    g = jnp.where(sel, jnp.where(acc >= 0, acc, 0.01 * acc) * 0.5, 0.0)
    out_ref[0] = g + g.T


def kernel(x):
    return pl.pallas_call(
        _graph_body,
        grid=(_B,),
        in_specs=[pl.BlockSpec((_T, 1, _N, _D), lambda b: (0, b, 0, 0))],
        out_specs=pl.BlockSpec((1, _N, _N), lambda b: (b, 0, 0)),
        out_shape=jax.ShapeDtypeStruct((_B, _N, _N), jnp.float32),
    )(x)
